# baseline (speedup 1.0000x reference)
import functools

import jax
import jax.numpy as jnp
from jax import lax
from jax.experimental import pallas as pl
from jax.experimental.pallas import tpu as pltpu
from jax.experimental.pallas import tpu_sc as plsc

N = 10000
E = 320000
D_IN = 128
H = 64
MW = 128

NC = 2
NS = 16
NW = NC * NS
CH = 64
EPT = E // NW
EPT_PAD = 10240
CPT = EPT_PAD // CH
PAD_PER_TILE = EPT_PAD - EPT
DUMP = 32
ROWS_PER_TILE = 624
REM_BASE = NS * ROWS_PER_TILE
REM = N - REM_BASE
IG = 16
NG = CPT // IG
NB = 4


_XP_NO_SCATTER = False


def _sc_agg_body(msg_hbm, src_hbm, dst_hbm, agg_out,
                 is0, is1, id0, id1, r0, r1, r2, r3, acc_sh,
                 gs0, gs1, gs2, gs3, ss0, ss1, ss2, ss3, isem0, isem1):
    isb = (is0, is1)
    idb = (id0, id1)
    rows = (r0, r1, r2, r3)
    gsems = (gs0, gs1, gs2, gs3)
    ssems = (ss0, ss1, ss2, ss3)
    isems = (isem0, isem1)

    c = lax.axis_index("c")
    s = lax.axis_index("s")
    wid = c * NS + s

    def start_idx(g, gb):
        off = pl.multiple_of(g * IG, 8)
        pltpu.async_copy(src_hbm.at[wid, pl.ds(off, IG)], isb[gb],
                         isems[gb])
        pltpu.async_copy(dst_hbm.at[wid, pl.ds(off, IG)], idb[gb],
                         isems[gb])

    def wait_idx(gb):
        pltpu.make_async_copy(src_hbm.at[wid, pl.ds(0, IG)], isb[gb],
                              isems[gb]).wait()
        pltpu.make_async_copy(dst_hbm.at[wid, pl.ds(0, IG)], idb[gb],
                              isems[gb]).wait()

    def start_gather(idx_row, b):
        pltpu.async_copy(msg_hbm.at[idx_row], rows[b], gsems[b])

    def wait_gather(b):
        pltpu.make_async_copy(msg_hbm.at[is0.at[0]], rows[b],
                              gsems[b]).wait()

    def start_scatter(idx_row, b):
        pltpu.async_copy(rows[b], acc_sh.at[idx_row], ssems[b], add=True)

    def wait_scatter(b):
        pltpu.make_async_copy(rows[b], acc_sh.at[id0.at[0]],
                              ssems[b]).wait()

    start_idx(0, 0)
    start_idx(1, 1)

    zeros16 = jnp.zeros((16,), jnp.float32)

    @pl.loop(0, CH)
    def _(i):
        for j in range(MW // 16):
            r0[i, pl.ds(j * 16, 16)] = zeros16

    base = pl.multiple_of(s * ROWS_PER_TILE, 8)
    for k in range(ROWS_PER_TILE // CH):
        pltpu.sync_copy(r0, acc_sh.at[pl.ds(base + k * CH, CH)])
    tail = ROWS_PER_TILE % CH
    if tail:
        pltpu.sync_copy(r0.at[pl.ds(0, tail)],
                        acc_sh.at[pl.ds(base + ROWS_PER_TILE - tail, tail)])

    @pl.when(s == 0)
    def _():
        pltpu.sync_copy(r0.at[pl.ds(0, REM)],
                        acc_sh.at[pl.ds(REM_BASE, REM)])
        pltpu.sync_copy(r0.at[pl.ds(0, DUMP)], acc_sh.at[pl.ds(N, DUMP)])

    plsc.subcore_barrier()

    wait_idx(0)
    start_gather(is0.at[0], 0)
    start_gather(is0.at[1], 1)
    start_gather(is0.at[2], 2)

    def do_chunk(g, gb, k, first_chunk=False, prologue_groups=False):
        def when(cond, fn):
            if isinstance(cond, bool):
                if cond:
                    fn()
            else:
                pl.when(cond)(fn)

        cch = g * IG + k
        r = k % NB
        nr = (k + 3) % NB
        if not first_chunk and not _XP_NO_SCATTER:
            wait_scatter(nr)
        tgt_gb = gb if k < IG - 3 else gb ^ 1
        slot = (k + 3) % IG
        when(cch + 3 < CPT,
             lambda: start_gather(isb[tgt_gb].at[slot], nr))
        wait_gather(r)
        if not _XP_NO_SCATTER:
            start_scatter(idb[gb].at[k], r)
        if k == 4 and not prologue_groups:
            when(g + 1 < NG, lambda: start_idx(g + 1, gb ^ 1))
        if k == 13:
            when(g + 1 < NG, lambda: wait_idx(gb ^ 1))

    for k in range(IG):
        do_chunk(0, 0, k, first_chunk=(k == 0), prologue_groups=True)
    for k in range(IG):
        do_chunk(1, 1, k)

    @pl.loop(2, NG, step=2)
    def _(go):
        for gb in range(2):
            for k in range(IG):
                do_chunk(go + gb, gb, k)

    if not _XP_NO_SCATTER:
        wait_scatter((CPT - 1) % NB)

    plsc.subcore_barrier()

    pltpu.sync_copy(acc_sh.at[pl.ds(base, ROWS_PER_TILE)],
                    agg_out.at[c, pl.ds(base, ROWS_PER_TILE)])

    @pl.when(s == 0)
    def _():
        pltpu.sync_copy(acc_sh.at[pl.ds(REM_BASE, REM)],
                        agg_out.at[c, pl.ds(REM_BASE, REM)])


@functools.lru_cache(maxsize=None)
def _sc_agg():
    mesh = plsc.VectorSubcoreMesh(core_axis_name="c", subcore_axis_name="s")
    return pl.kernel(
        _sc_agg_body,
        out_type=jax.ShapeDtypeStruct((NC, N, MW), jnp.float32),
        mesh=mesh,
        scratch_types=(
            [pltpu.VMEM((IG, CH), jnp.int32)] * 4 +
            [pltpu.VMEM((CH, MW), jnp.float32)] * NB +
            [pltpu.VMEM_SHARED((N + DUMP, MW), jnp.float32)] +
            [pltpu.SemaphoreType.DMA] * (2 * NB + 2)
        ),
    )


_CT11 = (((1,), (1,)), ((), ()))

BN = 1000
GRID = (N // BN,)


def _mm(a, w):
    return lax.dot_general(a, w, _CT11, preferred_element_type=jnp.float32)


def _msg_fill(h, wl_ref):
    hl = _mm(h, wl_ref[...])
    ones = jnp.ones((hl.shape[0], 16), jnp.float32)
    zeros = jnp.zeros((hl.shape[0], MW - H - 16), jnp.float32)
    return jnp.concatenate([hl, ones, zeros], axis=1)


def _tc0_body(x_ref, wl_ref, wr_ref, b_ref, msg_ref, hr_ref):
    x = x_ref[...]
    msg_ref[...] = _msg_fill(x, wl_ref)
    hr_ref[...] = _mm(x, wr_ref[...]) + b_ref[...]


def _combine(p_ref, hr_ref):
    p = p_ref[0] + p_ref[1]
    deg = p[:, H:H + 1]
    inv = 1.0 / jnp.maximum(deg, 1.0)
    return jnp.maximum(p[:, :H] * inv + hr_ref[...], 0.0)


def _tcmid_body(p_ref, hr_ref, wl_ref, wr_ref, b_ref, msg_ref, hrn_ref):
    h = _combine(p_ref, hr_ref)
    msg_ref[...] = _msg_fill(h, wl_ref)
    hrn_ref[...] = _mm(h, wr_ref[...]) + b_ref[...]


def _tcfin_body(p_ref, hr_ref, wh_ref, bh_ref, out_ref):
    h = _combine(p_ref, hr_ref)
    out_ref[...] = _mm(h, wh_ref[...]) + bh_ref[...]


def _row_spec(w):
    return pl.BlockSpec((BN, w), lambda i: (i, 0))


def _full_spec(shape):
    nd = len(shape)
    return pl.BlockSpec(shape, lambda i: (0,) * nd)


_P_SPEC = pl.BlockSpec((NC, BN, MW), lambda i: (0, i, 0))


@functools.lru_cache(maxsize=None)
def _tc0():
    return pl.pallas_call(
        _tc0_body,
        grid=GRID,
        in_specs=[_row_spec(D_IN), _full_spec((H, D_IN)),
                  _full_spec((H, D_IN)), _full_spec((1, H))],
        out_specs=[_row_spec(MW), _row_spec(H)],
        out_shape=[jax.ShapeDtypeStruct((N, MW), jnp.float32),
                   jax.ShapeDtypeStruct((N, H), jnp.float32)],
    )


@functools.lru_cache(maxsize=None)
def _tcmid():
    return pl.pallas_call(
        _tcmid_body,
        grid=GRID,
        in_specs=[_P_SPEC, _row_spec(H), _full_spec((H, H)),
                  _full_spec((H, H)), _full_spec((1, H))],
        out_specs=[_row_spec(MW), _row_spec(H)],
        out_shape=[jax.ShapeDtypeStruct((N, MW), jnp.float32),
                   jax.ShapeDtypeStruct((N, H), jnp.float32)],
    )


@functools.lru_cache(maxsize=None)
def _tcfin():
    return pl.pallas_call(
        _tcfin_body,
        grid=GRID,
        in_specs=[_P_SPEC, _row_spec(H), _full_spec((MW, H)),
                  _full_spec((1, MW))],
        out_specs=_row_spec(MW),
        out_shape=jax.ShapeDtypeStruct((N, MW), jnp.float32),
    )


def _pad_indices(idx, pad_base):
    tiles = idx.reshape(NW, EPT)
    pad_row = (jnp.arange(PAD_PER_TILE, dtype=jnp.int32) % DUMP) + pad_base
    pad = jnp.broadcast_to(pad_row, (NW, PAD_PER_TILE))
    return jnp.concatenate([tiles, pad], axis=1).reshape(NW, CPT, CH)


@jax.jit
def kernel(x, edge_index, W1l, b1, W1r, W2l, b2, W2r, W3l, b3, W3r, Wh, bh):
    src = _pad_indices(edge_index[0].astype(jnp.int32), 0)
    dst = _pad_indices(edge_index[1].astype(jnp.int32), N)

    b1r = b1.reshape(1, H)
    b2r = b2.reshape(1, H)
    b3r = b3.reshape(1, H)
    whr = jnp.broadcast_to(Wh.reshape(1, H), (MW, H))
    bhr = jnp.broadcast_to(bh.reshape(1, 1), (1, MW))

    agg = _sc_agg()

    msg1, hr1 = _tc0()(x, W1l, W1r, b1r)
    p1 = agg(msg1, src, dst)
    msg2, hr2 = _tcmid()(p1, hr1, W2l, W2r, b2r)
    p2 = agg(msg2, src, dst)
    msg3, hr3 = _tcmid()(p2, hr2, W3l, W3r, b3r)
    p3 = agg(msg3, src, dst)
    logits = _tcfin()(p3, hr3, whr, bhr)
    return logits[:, 0]


# speedup vs baseline: 1.1499x; 1.1499x over previous
import functools

import jax
import jax.numpy as jnp
from jax import lax
from jax.experimental import pallas as pl
from jax.experimental.pallas import tpu as pltpu
from jax.experimental.pallas import tpu_sc as plsc

N = 10000
E = 320000
D_IN = 128
H = 64
MW = 128
AW = 80

NC = 2
NS = 16
NW = NC * NS
CH = 64
EPT = E // NW
EPT_PAD = 10240
CPT = EPT_PAD // CH
PAD_PER_TILE = EPT_PAD - EPT
DUMP = 32
ROWS_PER_TILE = 624
REM_BASE = NS * ROWS_PER_TILE
REM = N - REM_BASE
IG = 16
NG = CPT // IG
NB = 4


_XP_NO_SCATTER = False


def _sc_agg_body(msg_hbm, src_hbm, dst_hbm, agg_out,
                 is0, is1, id0, id1, r0, r1, r2, r3, acc_sh,
                 gs0, gs1, gs2, gs3, ss0, ss1, ss2, ss3, isem0, isem1):
    isb = (is0, is1)
    idb = (id0, id1)
    rows = (r0, r1, r2, r3)
    gsems = (gs0, gs1, gs2, gs3)
    ssems = (ss0, ss1, ss2, ss3)
    isems = (isem0, isem1)

    c = lax.axis_index("c")
    s = lax.axis_index("s")
    wid = c * NS + s

    def start_idx(g, gb):
        off = pl.multiple_of(g * IG, 8)
        pltpu.async_copy(src_hbm.at[wid, pl.ds(off, IG)], isb[gb],
                         isems[gb])
        pltpu.async_copy(dst_hbm.at[wid, pl.ds(off, IG)], idb[gb],
                         isems[gb])

    def wait_idx(gb):
        pltpu.make_async_copy(src_hbm.at[wid, pl.ds(0, IG)], isb[gb],
                              isems[gb]).wait()
        pltpu.make_async_copy(dst_hbm.at[wid, pl.ds(0, IG)], idb[gb],
                              isems[gb]).wait()

    def start_gather(idx_row, b):
        pltpu.async_copy(msg_hbm.at[idx_row], rows[b], gsems[b])

    def wait_gather(b):
        pltpu.make_async_copy(msg_hbm.at[is0.at[0]], rows[b],
                              gsems[b]).wait()

    def start_scatter(idx_row, b):
        pltpu.async_copy(rows[b], acc_sh.at[idx_row], ssems[b], add=True)

    def wait_scatter(b):
        pltpu.make_async_copy(rows[b], acc_sh.at[id0.at[0]],
                              ssems[b]).wait()

    start_idx(0, 0)
    start_idx(1, 1)

    zeros16 = jnp.zeros((16,), jnp.float32)

    @pl.loop(0, CH)
    def _(i):
        for j in range(AW // 16):
            r0[i, pl.ds(j * 16, 16)] = zeros16

    base = pl.multiple_of(s * ROWS_PER_TILE, 8)
    for k in range(ROWS_PER_TILE // CH):
        pltpu.sync_copy(r0, acc_sh.at[pl.ds(base + k * CH, CH)])
    tail = ROWS_PER_TILE % CH
    if tail:
        pltpu.sync_copy(r0.at[pl.ds(0, tail)],
                        acc_sh.at[pl.ds(base + ROWS_PER_TILE - tail, tail)])

    @pl.when(s == 0)
    def _():
        pltpu.sync_copy(r0.at[pl.ds(0, REM)],
                        acc_sh.at[pl.ds(REM_BASE, REM)])
        pltpu.sync_copy(r0.at[pl.ds(0, DUMP)], acc_sh.at[pl.ds(N, DUMP)])

    plsc.subcore_barrier()

    wait_idx(0)
    start_gather(is0.at[0], 0)
    start_gather(is0.at[1], 1)
    start_gather(is0.at[2], 2)

    def do_chunk(g, gb, k, first_chunk=False, prologue_groups=False):
        def when(cond, fn):
            if isinstance(cond, bool):
                if cond:
                    fn()
            else:
                pl.when(cond)(fn)

        cch = g * IG + k
        r = k % NB
        nr = (k + 3) % NB
        if not first_chunk and not _XP_NO_SCATTER:
            wait_scatter(nr)
        tgt_gb = gb if k < IG - 3 else gb ^ 1
        slot = (k + 3) % IG
        when(cch + 3 < CPT,
             lambda: start_gather(isb[tgt_gb].at[slot], nr))
        wait_gather(r)
        if not _XP_NO_SCATTER:
            start_scatter(idb[gb].at[k], r)
        if k == 4 and not prologue_groups:
            when(g + 1 < NG, lambda: start_idx(g + 1, gb ^ 1))
        if k == 13:
            when(g + 1 < NG, lambda: wait_idx(gb ^ 1))

    for k in range(IG):
        do_chunk(0, 0, k, first_chunk=(k == 0), prologue_groups=True)
    for k in range(IG):
        do_chunk(1, 1, k)

    @pl.loop(2, NG, step=2)
    def _(go):
        for gb in range(2):
            for k in range(IG):
                do_chunk(go + gb, gb, k)

    if not _XP_NO_SCATTER:
        wait_scatter((CPT - 1) % NB)

    plsc.subcore_barrier()

    pltpu.sync_copy(acc_sh.at[pl.ds(base, ROWS_PER_TILE)],
                    agg_out.at[c, pl.ds(base, ROWS_PER_TILE)])

    @pl.when(s == 0)
    def _():
        pltpu.sync_copy(acc_sh.at[pl.ds(REM_BASE, REM)],
                        agg_out.at[c, pl.ds(REM_BASE, REM)])


@functools.lru_cache(maxsize=None)
def _sc_agg():
    mesh = plsc.VectorSubcoreMesh(core_axis_name="c", subcore_axis_name="s")
    return pl.kernel(
        _sc_agg_body,
        out_type=jax.ShapeDtypeStruct((NC, N, AW), jnp.float32),
        mesh=mesh,
        compiler_params=pltpu.CompilerParams(use_tc_tiling_on_sc=False),
        scratch_types=(
            [pltpu.VMEM((IG, CH), jnp.int32)] * 4 +
            [pltpu.VMEM((CH, AW), jnp.float32)] * NB +
            [pltpu.VMEM_SHARED((N + DUMP, AW), jnp.float32)] +
            [pltpu.SemaphoreType.DMA] * (2 * NB + 2)
        ),
    )


_CT11 = (((1,), (1,)), ((), ()))

BN = 1000
GRID = (N // BN,)


def _mm(a, w):
    return lax.dot_general(a, w, _CT11, preferred_element_type=jnp.float32)


def _msg_fill(h, wl_ref):
    hl = _mm(h, wl_ref[...])
    ones = jnp.ones((hl.shape[0], AW - H), jnp.float32)
    return jnp.concatenate([hl, ones], axis=1)


def _tc0_body(x_ref, wl_ref, wr_ref, b_ref, msg_ref, hr_ref):
    x = x_ref[...]
    msg_ref[...] = _msg_fill(x, wl_ref)
    hr_ref[...] = _mm(x, wr_ref[...]) + b_ref[...]


def _combine(p_ref, hr_ref):
    p = p_ref[0] + p_ref[1]
    deg = p[:, H:H + 1]
    inv = 1.0 / jnp.maximum(deg, 1.0)
    return jnp.maximum(p[:, :H] * inv + hr_ref[...], 0.0)


def _tcmid_body(p_ref, hr_ref, wl_ref, wr_ref, b_ref, msg_ref, hrn_ref):
    h = _combine(p_ref, hr_ref)
    msg_ref[...] = _msg_fill(h, wl_ref)
    hrn_ref[...] = _mm(h, wr_ref[...]) + b_ref[...]


def _tcfin_body(p_ref, hr_ref, wh_ref, bh_ref, out_ref):
    h = _combine(p_ref, hr_ref)
    out_ref[...] = _mm(h, wh_ref[...]) + bh_ref[...]


def _row_spec(w):
    return pl.BlockSpec((BN, w), lambda i: (i, 0))


def _full_spec(shape):
    nd = len(shape)
    return pl.BlockSpec(shape, lambda i: (0,) * nd)


_P_SPEC = pl.BlockSpec((NC, BN, AW), lambda i: (0, i, 0))


@functools.lru_cache(maxsize=None)
def _tc0():
    return pl.pallas_call(
        _tc0_body,
        grid=GRID,
        in_specs=[_row_spec(D_IN), _full_spec((H, D_IN)),
                  _full_spec((H, D_IN)), _full_spec((1, H))],
        out_specs=[_row_spec(AW), _row_spec(H)],
        out_shape=[jax.ShapeDtypeStruct((N, AW), jnp.float32),
                   jax.ShapeDtypeStruct((N, H), jnp.float32)],
    )


@functools.lru_cache(maxsize=None)
def _tcmid():
    return pl.pallas_call(
        _tcmid_body,
        grid=GRID,
        in_specs=[_P_SPEC, _row_spec(H), _full_spec((H, H)),
                  _full_spec((H, H)), _full_spec((1, H))],
        out_specs=[_row_spec(AW), _row_spec(H)],
        out_shape=[jax.ShapeDtypeStruct((N, AW), jnp.float32),
                   jax.ShapeDtypeStruct((N, H), jnp.float32)],
    )


@functools.lru_cache(maxsize=None)
def _tcfin():
    return pl.pallas_call(
        _tcfin_body,
        grid=GRID,
        in_specs=[_P_SPEC, _row_spec(H), _full_spec((MW, H)),
                  _full_spec((1, MW))],
        out_specs=_row_spec(MW),
        out_shape=jax.ShapeDtypeStruct((N, MW), jnp.float32),
    )


def _pad_indices(idx, pad_base):
    tiles = idx.reshape(NW, EPT)
    pad_row = (jnp.arange(PAD_PER_TILE, dtype=jnp.int32) % DUMP) + pad_base
    pad = jnp.broadcast_to(pad_row, (NW, PAD_PER_TILE))
    return jnp.concatenate([tiles, pad], axis=1).reshape(NW, CPT, CH)


@jax.jit
def kernel(x, edge_index, W1l, b1, W1r, W2l, b2, W2r, W3l, b3, W3r, Wh, bh):
    src = _pad_indices(edge_index[0].astype(jnp.int32), 0)
    dst = _pad_indices(edge_index[1].astype(jnp.int32), N)

    b1r = b1.reshape(1, H)
    b2r = b2.reshape(1, H)
    b3r = b3.reshape(1, H)
    whr = jnp.broadcast_to(Wh.reshape(1, H), (MW, H))
    bhr = jnp.broadcast_to(bh.reshape(1, 1), (1, MW))

    agg = _sc_agg()

    msg1, hr1 = _tc0()(x, W1l, W1r, b1r)
    p1 = agg(msg1, src, dst)
    msg2, hr2 = _tcmid()(p1, hr1, W2l, W2r, b2r)
    p2 = agg(msg2, src, dst)
    msg3, hr3 = _tcmid()(p2, hr2, W3l, W3r, b3r)
    p3 = agg(msg3, src, dst)
    logits = _tcfin()(p3, hr3, whr, bhr)
    return logits[:, 0]


# speedup vs baseline: 1.3147x; 1.1433x over previous
import functools

import jax
import jax.numpy as jnp
from jax import lax
from jax.experimental import pallas as pl
from jax.experimental.pallas import tpu as pltpu
from jax.experimental.pallas import tpu_sc as plsc

N = 10000
E = 320000
D_IN = 128
H = 64
MW = 128
AW = 80

NC = 2
NS = 16
NW = NC * NS
CH = 80
EPT = E // NW
CPT = EPT // CH
ROWS_PER_TILE = 624
REM_BASE = NS * ROWS_PER_TILE
REM = N - REM_BASE
IG = 25
NG = CPT // IG
NB = 4


def _sc_agg_body(msg_hbm, ei_hbm, agg_out,
                 is0, is1, id0, id1, r0, r1, r2, r3, acc_sh,
                 gs0, gs1, gs2, gs3, ss0, ss1, ss2, ss3, isem0, isem1):
    isb = (is0, is1)
    idb = (id0, id1)
    rows = (r0, r1, r2, r3)
    gsems = (gs0, gs1, gs2, gs3)
    ssems = (ss0, ss1, ss2, ss3)
    isems = (isem0, isem1)

    c = lax.axis_index("c")
    s = lax.axis_index("s")
    wid = c * NS + s

    def start_idx(g, gb):
        off = g * IG
        pltpu.async_copy(ei_hbm.at[0, wid, pl.ds(off, IG)], isb[gb],
                         isems[gb])
        pltpu.async_copy(ei_hbm.at[1, wid, pl.ds(off, IG)], idb[gb],
                         isems[gb])

    def wait_idx(gb):
        pltpu.make_async_copy(ei_hbm.at[0, wid, pl.ds(0, IG)], isb[gb],
                              isems[gb]).wait()
        pltpu.make_async_copy(ei_hbm.at[1, wid, pl.ds(0, IG)], idb[gb],
                              isems[gb]).wait()

    def start_gather(idx_row, b):
        pltpu.async_copy(msg_hbm.at[idx_row], rows[b], gsems[b])

    def wait_gather(b):
        pltpu.make_async_copy(msg_hbm.at[is0.at[0]], rows[b],
                              gsems[b]).wait()

    def start_scatter(idx_row, b):
        pltpu.async_copy(rows[b], acc_sh.at[idx_row], ssems[b], add=True)

    def wait_scatter(b):
        pltpu.make_async_copy(rows[b], acc_sh.at[id0.at[0]],
                              ssems[b]).wait()

    start_idx(0, 0)
    start_idx(1, 1)

    zeros16 = jnp.zeros((16,), jnp.float32)

    @pl.loop(0, CH)
    def _(i):
        for j in range(AW // 16):
            r0[i, pl.ds(j * 16, 16)] = zeros16

    base = pl.multiple_of(s * ROWS_PER_TILE, 8)
    for k in range(ROWS_PER_TILE // CH):
        pltpu.sync_copy(r0, acc_sh.at[pl.ds(base + k * CH, CH)])
    tail = ROWS_PER_TILE % CH
    if tail:
        pltpu.sync_copy(r0.at[pl.ds(0, tail)],
                        acc_sh.at[pl.ds(base + ROWS_PER_TILE - tail, tail)])

    @pl.when(s == 0)
    def _():
        pltpu.sync_copy(r0.at[pl.ds(0, REM)],
                        acc_sh.at[pl.ds(REM_BASE, REM)])

    plsc.subcore_barrier()

    wait_idx(0)
    start_gather(is0.at[0], 0)
    start_gather(is0.at[1], 1)
    start_gather(is0.at[2], 2)

    for ci in range(CPT):
        g, k = divmod(ci, IG)
        gb = g % 2
        r = ci % NB
        nr = (ci + 3) % NB
        if ci > 0:
            wait_scatter(nr)
        if k == 22 and g + 1 < NG:
            wait_idx(gb ^ 1)
        if ci + 3 < CPT:
            g3, slot = divmod(ci + 3, IG)
            start_gather(isb[g3 % 2].at[slot], nr)
        wait_gather(r)
        start_scatter(idb[gb].at[k], r)
        if k == 1 and 1 <= g < NG - 1:
            start_idx(g + 1, gb ^ 1)

    wait_scatter((CPT - 1) % NB)

    plsc.subcore_barrier()

    pltpu.sync_copy(acc_sh.at[pl.ds(base, ROWS_PER_TILE)],
                    agg_out.at[c, pl.ds(base, ROWS_PER_TILE)])

    @pl.when(s == 0)
    def _():
        pltpu.sync_copy(acc_sh.at[pl.ds(REM_BASE, REM)],
                        agg_out.at[c, pl.ds(REM_BASE, REM)])


@functools.lru_cache(maxsize=None)
def _sc_agg():
    mesh = plsc.VectorSubcoreMesh(core_axis_name="c", subcore_axis_name="s")
    return pl.kernel(
        _sc_agg_body,
        out_type=jax.ShapeDtypeStruct((NC, N, AW), jnp.float32),
        mesh=mesh,
        compiler_params=pltpu.CompilerParams(use_tc_tiling_on_sc=False),
        scratch_types=(
            [pltpu.VMEM((IG, CH), jnp.int32)] * 4 +
            [pltpu.VMEM((CH, AW), jnp.float32)] * NB +
            [pltpu.VMEM_SHARED((N, AW), jnp.float32)] +
            [pltpu.SemaphoreType.DMA] * (2 * NB + 2)
        ),
    )


_CT11 = (((1,), (1,)), ((), ()))

BN = 1000
GRID = (N // BN,)


def _mm(a, w):
    return lax.dot_general(a, w, _CT11, preferred_element_type=jnp.float32)


def _msg_fill(h, wl_ref):
    hl = _mm(h, wl_ref[...])
    ones = jnp.ones((hl.shape[0], AW - H), jnp.float32)
    return jnp.concatenate([hl, ones], axis=1)


def _tc0_body(x_ref, wl_ref, wr_ref, b_ref, msg_ref, hr_ref):
    x = x_ref[...]
    msg_ref[...] = _msg_fill(x, wl_ref)
    hr_ref[...] = _mm(x, wr_ref[...]) + b_ref[...]


def _combine(p_ref, hr_ref):
    p = p_ref[0] + p_ref[1]
    deg = p[:, H:H + 1]
    inv = 1.0 / jnp.maximum(deg, 1.0)
    return jnp.maximum(p[:, :H] * inv + hr_ref[...], 0.0)


def _tcmid_body(p_ref, hr_ref, wl_ref, wr_ref, b_ref, msg_ref, hrn_ref):
    h = _combine(p_ref, hr_ref)
    msg_ref[...] = _msg_fill(h, wl_ref)
    hrn_ref[...] = _mm(h, wr_ref[...]) + b_ref[...]


def _tcfin_body(p_ref, hr_ref, wh_ref, bh_ref, out_ref):
    h = _combine(p_ref, hr_ref)
    out_ref[...] = _mm(h, wh_ref[...]) + bh_ref[...]


def _row_spec(w):
    return pl.BlockSpec((BN, w), lambda i: (i, 0))


def _full_spec(shape):
    nd = len(shape)
    return pl.BlockSpec(shape, lambda i: (0,) * nd)


_P_SPEC = pl.BlockSpec((NC, BN, AW), lambda i: (0, i, 0))


@functools.lru_cache(maxsize=None)
def _tc0():
    return pl.pallas_call(
        _tc0_body,
        grid=GRID,
        in_specs=[_row_spec(D_IN), _full_spec((H, D_IN)),
                  _full_spec((H, D_IN)), _full_spec((1, H))],
        out_specs=[_row_spec(AW), _row_spec(H)],
        out_shape=[jax.ShapeDtypeStruct((N, AW), jnp.float32),
                   jax.ShapeDtypeStruct((N, H), jnp.float32)],
    )


@functools.lru_cache(maxsize=None)
def _tcmid():
    return pl.pallas_call(
        _tcmid_body,
        grid=GRID,
        in_specs=[_P_SPEC, _row_spec(H), _full_spec((H, H)),
                  _full_spec((H, H)), _full_spec((1, H))],
        out_specs=[_row_spec(AW), _row_spec(H)],
        out_shape=[jax.ShapeDtypeStruct((N, AW), jnp.float32),
                   jax.ShapeDtypeStruct((N, H), jnp.float32)],
    )


@functools.lru_cache(maxsize=None)
def _tcfin():
    return pl.pallas_call(
        _tcfin_body,
        grid=GRID,
        in_specs=[_P_SPEC, _row_spec(H), _full_spec((MW, H)),
                  _full_spec((1, MW))],
        out_specs=_row_spec(MW),
        out_shape=jax.ShapeDtypeStruct((N, MW), jnp.float32),
    )


@jax.jit
def kernel(x, edge_index, W1l, b1, W1r, W2l, b2, W2r, W3l, b3, W3r, Wh, bh):
    ei4 = edge_index.astype(jnp.int32).reshape(2, NW, CPT, CH)

    b1r = b1.reshape(1, H)
    b2r = b2.reshape(1, H)
    b3r = b3.reshape(1, H)
    whr = jnp.broadcast_to(Wh.reshape(1, H), (MW, H))
    bhr = jnp.broadcast_to(bh.reshape(1, 1), (1, MW))

    agg = _sc_agg()

    msg1, hr1 = _tc0()(x, W1l, W1r, b1r)
    p1 = agg(msg1, ei4)
    msg2, hr2 = _tcmid()(p1, hr1, W2l, W2r, b2r)
    p2 = agg(msg2, ei4)
    msg3, hr3 = _tcmid()(p2, hr2, W3l, W3r, b3r)
    p3 = agg(msg3, ei4)
    logits = _tcfin()(p3, hr3, whr, bhr)
    return logits[:, 0]


# speedup vs baseline: 1.4237x; 1.0830x over previous
import functools

import jax
import jax.numpy as jnp
from jax import lax
from jax.experimental import pallas as pl
from jax.experimental.pallas import tpu as pltpu
from jax.experimental.pallas import tpu_sc as plsc

N = 10000
E = 320000
D_IN = 128
H = 64
MW = 128
AW = 64
DW = 16

NC = 2
NS = 16
NW = NC * NS
CH = 80
EPT = E // NW
CPT = EPT // CH
ROWS_PER_TILE = 624
REM_BASE = NS * ROWS_PER_TILE
REM = N - REM_BASE
IG = 25
NG = CPT // IG
NB = 4


def _sc_agg_body(with_deg, msg_hbm, ei_hbm, *rest):
    if with_deg:
        (agg_out, deg_out, is0, is1, id0, id1, r0, r1, r2, r3,
         ones_v, dz, acc_sh, dacc_sh,
         gs0, gs1, gs2, gs3, ss0, ss1, ss2, ss3, isem0, isem1,
         ds0, ds1) = rest
    else:
        (agg_out, is0, is1, id0, id1, r0, r1, r2, r3, acc_sh,
         gs0, gs1, gs2, gs3, ss0, ss1, ss2, ss3, isem0, isem1) = rest
    isb = (is0, is1)
    idb = (id0, id1)
    rows = (r0, r1, r2, r3)
    gsems = (gs0, gs1, gs2, gs3)
    ssems = (ss0, ss1, ss2, ss3)
    isems = (isem0, isem1)
    if with_deg:
        dsems = (ds0, ds1)

    c = lax.axis_index("c")
    s = lax.axis_index("s")
    wid = c * NS + s

    def start_idx(g, gb):
        off = g * IG
        pltpu.async_copy(ei_hbm.at[0, wid, pl.ds(off, IG)], isb[gb],
                         isems[gb])
        pltpu.async_copy(ei_hbm.at[1, wid, pl.ds(off, IG)], idb[gb],
                         isems[gb])

    def wait_idx(gb):
        pltpu.make_async_copy(ei_hbm.at[0, wid, pl.ds(0, IG)], isb[gb],
                              isems[gb]).wait()
        pltpu.make_async_copy(ei_hbm.at[1, wid, pl.ds(0, IG)], idb[gb],
                              isems[gb]).wait()

    def start_gather(idx_row, b):
        pltpu.async_copy(msg_hbm.at[idx_row], rows[b], gsems[b])

    def wait_gather(b):
        pltpu.make_async_copy(msg_hbm.at[is0.at[0]], rows[b],
                              gsems[b]).wait()

    def start_scatter(idx_row, b):
        pltpu.async_copy(rows[b], acc_sh.at[idx_row], ssems[b], add=True)

    def wait_scatter(b):
        pltpu.make_async_copy(rows[b], acc_sh.at[id0.at[0]],
                              ssems[b]).wait()

    if with_deg:
        def start_dscatter(idx_row, b):
            pltpu.async_copy(ones_v, dacc_sh.at[idx_row], dsems[b],
                             add=True)

        def wait_dscatter(b):
            pltpu.make_async_copy(ones_v, dacc_sh.at[id0.at[0]],
                                  dsems[b]).wait()

    start_idx(0, 0)
    start_idx(1, 1)

    zeros16 = jnp.zeros((16,), jnp.float32)

    @pl.loop(0, CH)
    def _(i):
        for j in range(AW // 16):
            r0[i, pl.ds(j * 16, 16)] = zeros16

    if with_deg:
        ones16 = jnp.ones((16,), jnp.float32)

        @pl.loop(0, CH)
        def _(i):
            ones_v[i, :] = ones16
            dz[i, :] = zeros16

    base = pl.multiple_of(s * ROWS_PER_TILE, 8)
    for k in range(ROWS_PER_TILE // CH):
        pltpu.sync_copy(r0, acc_sh.at[pl.ds(base + k * CH, CH)])
        if with_deg:
            pltpu.sync_copy(dz, dacc_sh.at[pl.ds(base + k * CH, CH)])
    tail = ROWS_PER_TILE % CH
    if tail:
        pltpu.sync_copy(r0.at[pl.ds(0, tail)],
                        acc_sh.at[pl.ds(base + ROWS_PER_TILE - tail, tail)])
        if with_deg:
            pltpu.sync_copy(
                dz.at[pl.ds(0, tail)],
                dacc_sh.at[pl.ds(base + ROWS_PER_TILE - tail, tail)])

    @pl.when(s == 0)
    def _():
        pltpu.sync_copy(r0.at[pl.ds(0, REM)],
                        acc_sh.at[pl.ds(REM_BASE, REM)])
        if with_deg:
            pltpu.sync_copy(dz.at[pl.ds(0, REM)],
                            dacc_sh.at[pl.ds(REM_BASE, REM)])

    plsc.subcore_barrier()

    wait_idx(0)
    start_gather(is0.at[0], 0)
    start_gather(is0.at[1], 1)
    start_gather(is0.at[2], 2)

    for ci in range(CPT):
        g, k = divmod(ci, IG)
        gb = g % 2
        r = ci % NB
        nr = (ci + 3) % NB
        if ci > 0:
            wait_scatter(nr)
        if k == 22 and g + 1 < NG:
            wait_idx(gb ^ 1)
        if ci + 3 < CPT:
            g3, slot = divmod(ci + 3, IG)
            start_gather(isb[g3 % 2].at[slot], nr)
        wait_gather(r)
        start_scatter(idb[gb].at[k], r)
        if with_deg:
            if ci >= 2:
                wait_dscatter(ci % 2)
            start_dscatter(idb[gb].at[k], ci % 2)
        if k == 1 and 1 <= g < NG - 1:
            start_idx(g + 1, gb ^ 1)

    wait_scatter((CPT - 1) % NB)
    if with_deg:
        wait_dscatter(0)
        wait_dscatter(1)

    plsc.subcore_barrier()

    pltpu.sync_copy(acc_sh.at[pl.ds(base, ROWS_PER_TILE)],
                    agg_out.at[c, pl.ds(base, ROWS_PER_TILE)])
    if with_deg:
        pltpu.sync_copy(dacc_sh.at[pl.ds(base, ROWS_PER_TILE)],
                        deg_out.at[c, pl.ds(base, ROWS_PER_TILE)])

    @pl.when(s == 0)
    def _():
        pltpu.sync_copy(acc_sh.at[pl.ds(REM_BASE, REM)],
                        agg_out.at[c, pl.ds(REM_BASE, REM)])
        if with_deg:
            pltpu.sync_copy(dacc_sh.at[pl.ds(REM_BASE, REM)],
                            deg_out.at[c, pl.ds(REM_BASE, REM)])


@functools.lru_cache(maxsize=None)
def _sc_agg(with_deg):
    mesh = plsc.VectorSubcoreMesh(core_axis_name="c", subcore_axis_name="s")
    out_type = [jax.ShapeDtypeStruct((NC, N, AW), jnp.float32)]
    if with_deg:
        out_type.append(jax.ShapeDtypeStruct((NC, N, DW), jnp.float32))
    scratch = [pltpu.VMEM((IG, CH), jnp.int32)] * 4
    scratch += [pltpu.VMEM((CH, AW), jnp.float32)] * NB
    if with_deg:
        scratch += [pltpu.VMEM((CH, DW), jnp.float32)] * 2
    scratch += [pltpu.VMEM_SHARED((N, AW), jnp.float32)]
    if with_deg:
        scratch += [pltpu.VMEM_SHARED((N, DW), jnp.float32)]
    scratch += [pltpu.SemaphoreType.DMA] * (2 * NB + 2)
    if with_deg:
        scratch += [pltpu.SemaphoreType.DMA] * 2
    return pl.kernel(
        functools.partial(_sc_agg_body, with_deg),
        out_type=out_type if with_deg else out_type[0],
        mesh=mesh,
        compiler_params=pltpu.CompilerParams(use_tc_tiling_on_sc=False),
        scratch_types=scratch,
    )


_CT11 = (((1,), (1,)), ((), ()))

BN = 1000
GRID = (N // BN,)


def _mm(a, w):
    return lax.dot_general(a, w, _CT11, preferred_element_type=jnp.float32)


def _tc0_body(x_ref, wl_ref, wr_ref, b_ref, msg_ref, hr_ref):
    x = x_ref[...]
    msg_ref[...] = _mm(x, wl_ref[...])
    hr_ref[...] = _mm(x, wr_ref[...]) + b_ref[...]


def _combine(p_ref, d_ref, hr_ref):
    deg = d_ref[0][:, 0:1] + d_ref[1][:, 0:1]
    inv = 1.0 / jnp.maximum(deg, 1.0)
    return jnp.maximum((p_ref[0] + p_ref[1]) * inv + hr_ref[...], 0.0)


def _tcmid_body(p_ref, d_ref, hr_ref, wl_ref, wr_ref, b_ref,
                msg_ref, hrn_ref):
    h = _combine(p_ref, d_ref, hr_ref)
    msg_ref[...] = _mm(h, wl_ref[...])
    hrn_ref[...] = _mm(h, wr_ref[...]) + b_ref[...]


def _tcfin_body(p_ref, d_ref, hr_ref, wh_ref, bh_ref, out_ref):
    h = _combine(p_ref, d_ref, hr_ref)
    out_ref[...] = _mm(h, wh_ref[...]) + bh_ref[...]


def _row_spec(w):
    return pl.BlockSpec((BN, w), lambda i: (i, 0))


def _full_spec(shape):
    nd = len(shape)
    return pl.BlockSpec(shape, lambda i: (0,) * nd)


_P_SPEC = pl.BlockSpec((NC, BN, AW), lambda i: (0, i, 0))
_D_SPEC = pl.BlockSpec((NC, BN, DW), lambda i: (0, i, 0))


@functools.lru_cache(maxsize=None)
def _tc0():
    return pl.pallas_call(
        _tc0_body,
        grid=GRID,
        in_specs=[_row_spec(D_IN), _full_spec((H, D_IN)),
                  _full_spec((H, D_IN)), _full_spec((1, H))],
        out_specs=[_row_spec(AW), _row_spec(H)],
        out_shape=[jax.ShapeDtypeStruct((N, AW), jnp.float32),
                   jax.ShapeDtypeStruct((N, H), jnp.float32)],
    )


@functools.lru_cache(maxsize=None)
def _tcmid():
    return pl.pallas_call(
        _tcmid_body,
        grid=GRID,
        in_specs=[_P_SPEC, _D_SPEC, _row_spec(H), _full_spec((H, H)),
                  _full_spec((H, H)), _full_spec((1, H))],
        out_specs=[_row_spec(AW), _row_spec(H)],
        out_shape=[jax.ShapeDtypeStruct((N, AW), jnp.float32),
                   jax.ShapeDtypeStruct((N, H), jnp.float32)],
    )


@functools.lru_cache(maxsize=None)
def _tcfin():
    return pl.pallas_call(
        _tcfin_body,
        grid=GRID,
        in_specs=[_P_SPEC, _D_SPEC, _row_spec(H), _full_spec((MW, H)),
                  _full_spec((1, MW))],
        out_specs=_row_spec(MW),
        out_shape=jax.ShapeDtypeStruct((N, MW), jnp.float32),
    )


@jax.jit
def kernel(x, edge_index, W1l, b1, W1r, W2l, b2, W2r, W3l, b3, W3r, Wh, bh):
    ei4 = edge_index.astype(jnp.int32).reshape(2, NW, CPT, CH)

    b1r = b1.reshape(1, H)
    b2r = b2.reshape(1, H)
    b3r = b3.reshape(1, H)
    whr = jnp.broadcast_to(Wh.reshape(1, H), (MW, H))
    bhr = jnp.broadcast_to(bh.reshape(1, 1), (1, MW))

    msg1, hr1 = _tc0()(x, W1l, W1r, b1r)
    p1, d = _sc_agg(True)(msg1, ei4)
    msg2, hr2 = _tcmid()(p1, d, hr1, W2l, W2r, b2r)
    p2 = _sc_agg(False)(msg2, ei4)
    msg3, hr3 = _tcmid()(p2, d, hr2, W3l, W3r, b3r)
    p3 = _sc_agg(False)(msg3, ei4)
    logits = _tcfin()(p3, d, hr3, whr, bhr)
    return logits[:, 0]


# speedup vs baseline: 1.4737x; 1.0351x over previous
import functools

import jax
import jax.numpy as jnp
from jax import lax
from jax.experimental import pallas as pl
from jax.experimental.pallas import tpu as pltpu
from jax.experimental.pallas import tpu_sc as plsc

N = 10000
E = 320000
D_IN = 128
H = 64
MW = 128
AW = 64
DW = 16

NC = 2
NS = 16
NW = NC * NS
CH = 80
EPT = E // NW
CPT = EPT // CH
ROWS_PER_TILE = 624
REM_BASE = NS * ROWS_PER_TILE
REM = N - REM_BASE
IG = 25
NG = CPT // IG
NB = 4


def _sc_agg_body(with_deg, msg_hbm, ei_hbm, *rest):
    if with_deg:
        (agg_out, deg_out, is0, is1, id0, id1, r0, r1, r2, r3,
         ones_v, dz, acc_sh, dacc_sh,
         gs0, gs1, gs2, gs3, ss0, ss1, ss2, ss3, isem0, isem1,
         ds0, ds1) = rest
    else:
        (agg_out, is0, is1, id0, id1, r0, r1, r2, r3, acc_sh,
         gs0, gs1, gs2, gs3, ss0, ss1, ss2, ss3, isem0, isem1) = rest
    isb = (is0, is1)
    idb = (id0, id1)
    rows = (r0, r1, r2, r3)
    gsems = (gs0, gs1, gs2, gs3)
    ssems = (ss0, ss1, ss2, ss3)
    isems = (isem0, isem1)
    if with_deg:
        dsems = (ds0, ds1)

    c = lax.axis_index("c")
    s = lax.axis_index("s")
    wid = c * NS + s

    def start_idx(g, gb):
        off = g * IG
        pltpu.async_copy(ei_hbm.at[0, wid, pl.ds(off, IG)], isb[gb],
                         isems[gb])
        pltpu.async_copy(ei_hbm.at[1, wid, pl.ds(off, IG)], idb[gb],
                         isems[gb])

    def wait_idx(gb):
        pltpu.make_async_copy(ei_hbm.at[0, wid, pl.ds(0, IG)], isb[gb],
                              isems[gb]).wait()
        pltpu.make_async_copy(ei_hbm.at[1, wid, pl.ds(0, IG)], idb[gb],
                              isems[gb]).wait()

    def start_gather(idx_row, b):
        pltpu.async_copy(msg_hbm.at[idx_row], rows[b], gsems[b])

    def wait_gather(b):
        pltpu.make_async_copy(msg_hbm.at[is0.at[0]], rows[b],
                              gsems[b]).wait()

    def start_scatter(idx_row, b):
        pltpu.async_copy(rows[b], acc_sh.at[idx_row], ssems[b], add=True)

    def wait_scatter(b):
        pltpu.make_async_copy(rows[b], acc_sh.at[id0.at[0]],
                              ssems[b]).wait()

    if with_deg:
        def start_dscatter(idx_row, b):
            pltpu.async_copy(ones_v, dacc_sh.at[idx_row], dsems[b],
                             add=True)

        def wait_dscatter(b):
            pltpu.make_async_copy(ones_v, dacc_sh.at[id0.at[0]],
                                  dsems[b]).wait()

    start_idx(0, 0)
    start_idx(1, 1)

    zeros16 = jnp.zeros((16,), jnp.float32)

    @pl.loop(0, CH)
    def _(i):
        for j in range(AW // 16):
            r0[i, pl.ds(j * 16, 16)] = zeros16

    if with_deg:
        ones16 = jnp.ones((16,), jnp.float32)

        @pl.loop(0, CH)
        def _(i):
            ones_v[i, :] = ones16
            dz[i, :] = zeros16

    base = pl.multiple_of(s * ROWS_PER_TILE, 8)
    for k in range(ROWS_PER_TILE // CH):
        pltpu.sync_copy(r0, acc_sh.at[pl.ds(base + k * CH, CH)])
        if with_deg:
            pltpu.sync_copy(dz, dacc_sh.at[pl.ds(base + k * CH, CH)])
    tail = ROWS_PER_TILE % CH
    if tail:
        pltpu.sync_copy(r0.at[pl.ds(0, tail)],
                        acc_sh.at[pl.ds(base + ROWS_PER_TILE - tail, tail)])
        if with_deg:
            pltpu.sync_copy(
                dz.at[pl.ds(0, tail)],
                dacc_sh.at[pl.ds(base + ROWS_PER_TILE - tail, tail)])

    @pl.when(s == 0)
    def _():
        pltpu.sync_copy(r0.at[pl.ds(0, REM)],
                        acc_sh.at[pl.ds(REM_BASE, REM)])
        if with_deg:
            pltpu.sync_copy(dz.at[pl.ds(0, REM)],
                            dacc_sh.at[pl.ds(REM_BASE, REM)])

    plsc.subcore_barrier()

    wait_idx(0)
    start_gather(is0.at[0], 0)
    start_gather(is0.at[1], 1)
    start_gather(is0.at[2], 2)

    for ci in range(CPT):
        g, k = divmod(ci, IG)
        gb = g % 2
        r = ci % NB
        nr = (ci + 3) % NB
        if ci > 0:
            wait_scatter(nr)
        if k == 22 and g + 1 < NG:
            wait_idx(gb ^ 1)
        if ci + 3 < CPT:
            g3, slot = divmod(ci + 3, IG)
            start_gather(isb[g3 % 2].at[slot], nr)
        wait_gather(r)
        start_scatter(idb[gb].at[k], r)
        if with_deg:
            if ci >= 2:
                wait_dscatter(ci % 2)
            start_dscatter(idb[gb].at[k], ci % 2)
        if k == 1 and 1 <= g < NG - 1:
            start_idx(g + 1, gb ^ 1)

    wait_scatter((CPT - 1) % NB)
    if with_deg:
        wait_dscatter(0)
        wait_dscatter(1)

    plsc.subcore_barrier()

    pltpu.sync_copy(acc_sh.at[pl.ds(base, ROWS_PER_TILE)],
                    agg_out.at[c, pl.ds(base, ROWS_PER_TILE)])
    if with_deg:
        pltpu.sync_copy(dacc_sh.at[pl.ds(base, ROWS_PER_TILE)],
                        deg_out.at[c, pl.ds(base, ROWS_PER_TILE)])

    @pl.when(s == 0)
    def _():
        pltpu.sync_copy(acc_sh.at[pl.ds(REM_BASE, REM)],
                        agg_out.at[c, pl.ds(REM_BASE, REM)])
        if with_deg:
            pltpu.sync_copy(dacc_sh.at[pl.ds(REM_BASE, REM)],
                            deg_out.at[c, pl.ds(REM_BASE, REM)])


@functools.lru_cache(maxsize=None)
def _sc_agg(with_deg):
    mesh = plsc.VectorSubcoreMesh(core_axis_name="c", subcore_axis_name="s")
    out_type = [jax.ShapeDtypeStruct((NC, N, AW), jnp.float32)]
    if with_deg:
        out_type.append(jax.ShapeDtypeStruct((NC, N, DW), jnp.float32))
    scratch = [pltpu.VMEM((IG, CH), jnp.int32)] * 4
    scratch += [pltpu.VMEM((CH, AW), jnp.float32)] * NB
    if with_deg:
        scratch += [pltpu.VMEM((CH, DW), jnp.float32)] * 2
    scratch += [pltpu.VMEM_SHARED((N, AW), jnp.float32)]
    if with_deg:
        scratch += [pltpu.VMEM_SHARED((N, DW), jnp.float32)]
    scratch += [pltpu.SemaphoreType.DMA] * (2 * NB + 2)
    if with_deg:
        scratch += [pltpu.SemaphoreType.DMA] * 2
    return pl.kernel(
        functools.partial(_sc_agg_body, with_deg),
        out_type=out_type if with_deg else out_type[0],
        mesh=mesh,
        compiler_params=pltpu.CompilerParams(use_tc_tiling_on_sc=False),
        scratch_types=scratch,
    )


_CT11 = (((1,), (1,)), ((), ()))

BN = 2000
GRID = (N // BN,)


def _mm(a, w):
    return lax.dot_general(a, w, _CT11, preferred_element_type=jnp.float32)


def _tc0_body(x_ref, wl_ref, wr_ref, b_ref, msg_ref, hr_ref):
    x = x_ref[...]
    msg_ref[...] = _mm(x, wl_ref[...])
    hr_ref[...] = _mm(x, wr_ref[...]) + b_ref[...]


def _combine(p_ref, d_ref, hr_ref):
    deg = d_ref[0][:, 0:1] + d_ref[1][:, 0:1]
    inv = 1.0 / jnp.maximum(deg, 1.0)
    return jnp.maximum((p_ref[0] + p_ref[1]) * inv + hr_ref[...], 0.0)


def _tcmid_body(p_ref, d_ref, hr_ref, wl_ref, wr_ref, b_ref,
                msg_ref, hrn_ref):
    h = _combine(p_ref, d_ref, hr_ref)
    msg_ref[...] = _mm(h, wl_ref[...])
    hrn_ref[...] = _mm(h, wr_ref[...]) + b_ref[...]


def _tcfin_body(p_ref, d_ref, hr_ref, wh_ref, bh_ref, out_ref):
    h = _combine(p_ref, d_ref, hr_ref)
    out_ref[...] = _mm(h, wh_ref[...]) + bh_ref[...]


def _row_spec(w):
    return pl.BlockSpec((BN, w), lambda i: (i, 0))


def _full_spec(shape):
    nd = len(shape)
    return pl.BlockSpec(shape, lambda i: (0,) * nd)


_P_SPEC = pl.BlockSpec((NC, BN, AW), lambda i: (0, i, 0))
_D_SPEC = pl.BlockSpec((NC, BN, DW), lambda i: (0, i, 0))


@functools.lru_cache(maxsize=None)
def _tc0():
    return pl.pallas_call(
        _tc0_body,
        grid=GRID,
        in_specs=[_row_spec(D_IN), _full_spec((H, D_IN)),
                  _full_spec((H, D_IN)), _full_spec((1, H))],
        out_specs=[_row_spec(AW), _row_spec(H)],
        out_shape=[jax.ShapeDtypeStruct((N, AW), jnp.float32),
                   jax.ShapeDtypeStruct((N, H), jnp.float32)],
    )


@functools.lru_cache(maxsize=None)
def _tcmid():
    return pl.pallas_call(
        _tcmid_body,
        grid=GRID,
        in_specs=[_P_SPEC, _D_SPEC, _row_spec(H), _full_spec((H, H)),
                  _full_spec((H, H)), _full_spec((1, H))],
        out_specs=[_row_spec(AW), _row_spec(H)],
        out_shape=[jax.ShapeDtypeStruct((N, AW), jnp.float32),
                   jax.ShapeDtypeStruct((N, H), jnp.float32)],
    )


@functools.lru_cache(maxsize=None)
def _tcfin():
    return pl.pallas_call(
        _tcfin_body,
        grid=GRID,
        in_specs=[_P_SPEC, _D_SPEC, _row_spec(H), _full_spec((MW, H)),
                  _full_spec((1, MW))],
        out_specs=_row_spec(MW),
        out_shape=jax.ShapeDtypeStruct((N, MW), jnp.float32),
    )


@jax.jit
def kernel(x, edge_index, W1l, b1, W1r, W2l, b2, W2r, W3l, b3, W3r, Wh, bh):
    ei4 = edge_index.astype(jnp.int32).reshape(2, NW, CPT, CH)

    b1r = b1.reshape(1, H)
    b2r = b2.reshape(1, H)
    b3r = b3.reshape(1, H)
    whr = jnp.broadcast_to(Wh.reshape(1, H), (MW, H))
    bhr = jnp.broadcast_to(bh.reshape(1, 1), (1, MW))

    msg1, hr1 = _tc0()(x, W1l, W1r, b1r)
    p1, d = _sc_agg(True)(msg1, ei4)
    msg2, hr2 = _tcmid()(p1, d, hr1, W2l, W2r, b2r)
    p2 = _sc_agg(False)(msg2, ei4)
    msg3, hr3 = _tcmid()(p2, d, hr2, W3l, W3r, b3r)
    p3 = _sc_agg(False)(msg3, ei4)
    logits = _tcfin()(p3, d, hr3, whr, bhr)
    return logits[:, 0]
